# baseline (device time: 1740525 ns/iter reference)
import jax
import jax.numpy as jnp
from jax import lax
from jax.experimental import pallas as pl
from jax.experimental.pallas import tpu as pltpu

N = 32
BS = 512
D = 256
NHOP = N - 1


def kernel(x, Win0, Wout0, Win1, Wout1, Win2, Wout2):
    def body(x_ref, win0, wout0, win1, wout1, win2, wout2, out_ref,
             comm_x, comm_s, sx_send, sx_recv, ss_send, ss_recv):
        me = lax.axis_index("i")
        left = lax.rem(me + N - 1, N)
        right = lax.rem(me + 1, N)

        barrier = pltpu.get_barrier_semaphore()
        for nbr in (left, right):
            pl.semaphore_signal(barrier, inc=1, device_id=(nbr,),
                                device_id_type=pl.DeviceIdType.MESH)
        pl.semaphore_wait(barrier, 2)

        def mlp(xc, win, wout):
            h = jnp.maximum(
                jax.lax.dot(xc, win[...],
                            preferred_element_type=jnp.float32,
                            precision=jax.lax.Precision.HIGHEST),
                0.0)
            return jax.lax.dot(h, wout[...],
                               preferred_element_type=jnp.float32,
                               precision=jax.lax.Precision.HIGHEST)

        weights = ((win0, wout0), (win1, wout1), (win2, wout2))

        x_cur = x_ref[...]
        for l in range(3):
            win, wout = weights[l]
            ss0 = (l * NHOP) % 2
            s0 = mlp(x_cur, win, wout)
            comm_x[ss0] = x_cur
            comm_s[ss0] = s0

            def hop(s, _, l=l, win=win, wout=wout):
                g = l * NHOP + s
                ss = lax.rem(g, 2)
                rs = lax.rem(g + 1, 2)
                rx = pltpu.make_async_remote_copy(
                    src_ref=comm_x.at[ss], dst_ref=comm_x.at[rs],
                    send_sem=sx_send.at[ss], recv_sem=sx_recv.at[rs],
                    device_id=(right,), device_id_type=pl.DeviceIdType.MESH)
                rp = pltpu.make_async_remote_copy(
                    src_ref=comm_s.at[ss], dst_ref=comm_s.at[rs],
                    send_sem=ss_send.at[ss], recv_sem=ss_recv.at[rs],
                    device_id=(right,), device_id_type=pl.DeviceIdType.MESH)
                rx.start()
                rp.start()
                rx.wait()
                rp.wait()
                p = mlp(comm_x[rs], win, wout)
                comm_s[rs] = comm_s[rs] + p
                return 0

            lax.fori_loop(0, NHOP, hop, 0)
            x_cur = comm_s[((l + 1) * NHOP) % 2]

        cm = lax.rem(me + 3, N)
        out_ref[pl.ds(cm * BS, BS), :] = comm_s[1]

        def ag(s, _):
            g = 3 * NHOP + s
            ss = lax.rem(g, 2)
            rs = lax.rem(g + 1, 2)
            r = pltpu.make_async_remote_copy(
                src_ref=comm_s.at[ss], dst_ref=comm_s.at[rs],
                send_sem=ss_send.at[ss], recv_sem=ss_recv.at[rs],
                device_id=(right,), device_id_type=pl.DeviceIdType.MESH)
            r.start()
            r.wait()
            c = lax.rem(me + 2 - s + 2 * N, N)
            out_ref[pl.ds(c * BS, BS), :] = comm_s[rs]
            return 0

        lax.fori_loop(0, NHOP, ag, 0)

    return pl.pallas_call(
        body,
        out_shape=jax.ShapeDtypeStruct((N * BS, D), jnp.float32),
        in_specs=[pl.BlockSpec(memory_space=pltpu.VMEM)] * 7,
        out_specs=pl.BlockSpec(memory_space=pltpu.VMEM),
        scratch_shapes=[
            pltpu.VMEM((2, BS, D), jnp.float32),
            pltpu.VMEM((2, BS, D), jnp.float32),
            pltpu.SemaphoreType.DMA((2,)),
            pltpu.SemaphoreType.DMA((2,)),
            pltpu.SemaphoreType.DMA((2,)),
            pltpu.SemaphoreType.DMA((2,)),
        ],
        compiler_params=pltpu.CompilerParams(collective_id=0),
    )(x, Win0, Wout0, Win1, Wout1, Win2, Wout2)


# device time: 1681191 ns/iter; 1.0353x vs baseline; 1.0353x over previous
import jax
import jax.numpy as jnp
from jax import lax
from jax.experimental import pallas as pl
from jax.experimental.pallas import tpu as pltpu

N = 32
BS = 512
HB = 256
D = 256
NHOP = N - 1


def kernel(x, Win0, Wout0, Win1, Wout1, Win2, Wout2):
    def body(x_ref, win0, wout0, win1, wout1, win2, wout2, out_ref,
             cxa, csa, cxb, csb,
             sxa_s, sxa_r, ssa_s, ssa_r,
             sxb_s, sxb_r, ssb_s, ssb_r):
        me = lax.axis_index("i")
        left = lax.rem(me + N - 1, N)
        right = lax.rem(me + 1, N)

        barrier = pltpu.get_barrier_semaphore()
        for nbr in (left, right):
            pl.semaphore_signal(barrier, inc=1, device_id=(nbr,),
                                device_id_type=pl.DeviceIdType.MESH)
        pl.semaphore_wait(barrier, 2)

        def mlp(xc, win, wout):
            h = jnp.maximum(
                jax.lax.dot(xc, win[...],
                            preferred_element_type=jnp.float32,
                            precision=jax.lax.Precision.HIGHEST),
                0.0)
            return jax.lax.dot(h, wout[...],
                               preferred_element_type=jnp.float32,
                               precision=jax.lax.Precision.HIGHEST)

        weights = ((win0, wout0), (win1, wout1), (win2, wout2))

        xa = x_ref[0:HB, :]
        xb = x_ref[HB:BS, :]
        for l in range(3):
            win, wout = weights[l]
            ss0 = (l * NHOP) % 2
            p0 = mlp(jnp.concatenate([xa, xb], axis=0), win, wout)
            cxa[ss0] = xa
            csa[ss0] = p0[0:HB, :]
            cxb[ss0] = xb
            csb[ss0] = p0[HB:BS, :]

            def hop(s, _, win=win, wout=wout, l=l):
                g = l * NHOP + s
                ss = lax.rem(g, 2)
                rs = lax.rem(g + 1, 2)
                rdmas = [
                    pltpu.make_async_remote_copy(
                        src_ref=cxa.at[ss], dst_ref=cxa.at[rs],
                        send_sem=sxa_s.at[ss], recv_sem=sxa_r.at[rs],
                        device_id=(right,), device_id_type=pl.DeviceIdType.MESH),
                    pltpu.make_async_remote_copy(
                        src_ref=csa.at[ss], dst_ref=csa.at[rs],
                        send_sem=ssa_s.at[ss], recv_sem=ssa_r.at[rs],
                        device_id=(right,), device_id_type=pl.DeviceIdType.MESH),
                    pltpu.make_async_remote_copy(
                        src_ref=cxb.at[ss], dst_ref=cxb.at[rs],
                        send_sem=sxb_s.at[ss], recv_sem=sxb_r.at[rs],
                        device_id=(left,), device_id_type=pl.DeviceIdType.MESH),
                    pltpu.make_async_remote_copy(
                        src_ref=csb.at[ss], dst_ref=csb.at[rs],
                        send_sem=ssb_s.at[ss], recv_sem=ssb_r.at[rs],
                        device_id=(left,), device_id_type=pl.DeviceIdType.MESH),
                ]
                for r in rdmas:
                    r.start()
                for r in rdmas:
                    r.wait()
                p = mlp(jnp.concatenate([cxa[rs], cxb[rs]], axis=0), win, wout)
                csa[rs] = csa[rs] + p[0:HB, :]
                csb[rs] = csb[rs] + p[HB:BS, :]
                return 0

            lax.fori_loop(0, NHOP, hop, 0)
            xa = csa[((l + 1) * NHOP) % 2]
            xb = csb[((l + 1) * NHOP) % 2]

        cma = lax.rem(me + 3, N)
        cmb = lax.rem(me - 3 + N, N)
        out_ref[pl.ds(cma * BS, HB), :] = csa[1]
        out_ref[pl.ds(cmb * BS + HB, HB), :] = csb[1]

        def ag(s, _):
            g = 3 * NHOP + s
            ss = lax.rem(g, 2)
            rs = lax.rem(g + 1, 2)
            rdmas = [
                pltpu.make_async_remote_copy(
                    src_ref=csa.at[ss], dst_ref=csa.at[rs],
                    send_sem=ssa_s.at[ss], recv_sem=ssa_r.at[rs],
                    device_id=(right,), device_id_type=pl.DeviceIdType.MESH),
                pltpu.make_async_remote_copy(
                    src_ref=csb.at[ss], dst_ref=csb.at[rs],
                    send_sem=ssb_s.at[ss], recv_sem=ssb_r.at[rs],
                    device_id=(left,), device_id_type=pl.DeviceIdType.MESH),
            ]
            for r in rdmas:
                r.start()
            for r in rdmas:
                r.wait()
            ca = lax.rem(me + 2 - s + 2 * N, N)
            cb = lax.rem(me + s - 2 + 2 * N, N)
            out_ref[pl.ds(ca * BS, HB), :] = csa[rs]
            out_ref[pl.ds(cb * BS + HB, HB), :] = csb[rs]
            return 0

        lax.fori_loop(0, NHOP, ag, 0)

    return pl.pallas_call(
        body,
        out_shape=jax.ShapeDtypeStruct((N * BS, D), jnp.float32),
        in_specs=[pl.BlockSpec(memory_space=pltpu.VMEM)] * 7,
        out_specs=pl.BlockSpec(memory_space=pltpu.VMEM),
        scratch_shapes=[
            pltpu.VMEM((2, HB, D), jnp.float32),
            pltpu.VMEM((2, HB, D), jnp.float32),
            pltpu.VMEM((2, HB, D), jnp.float32),
            pltpu.VMEM((2, HB, D), jnp.float32),
            pltpu.SemaphoreType.DMA((2,)),
            pltpu.SemaphoreType.DMA((2,)),
            pltpu.SemaphoreType.DMA((2,)),
            pltpu.SemaphoreType.DMA((2,)),
            pltpu.SemaphoreType.DMA((2,)),
            pltpu.SemaphoreType.DMA((2,)),
            pltpu.SemaphoreType.DMA((2,)),
            pltpu.SemaphoreType.DMA((2,)),
        ],
        compiler_params=pltpu.CompilerParams(collective_id=0),
    )(x, Win0, Wout0, Win1, Wout1, Win2, Wout2)


# device time: 690218 ns/iter; 2.5217x vs baseline; 2.4357x over previous
import jax
import jax.numpy as jnp
from jax import lax
from jax.experimental import pallas as pl
from jax.experimental.pallas import tpu as pltpu

N = 32
BS = 512
HB = 256
D = 256
NHOP = N - 1
K = 4

MESH = pl.DeviceIdType.MESH

CYC = (0, 3, 4, 7, 15, 12, 11, 8, 16, 19, 20, 23, 31, 28, 27, 24,
       25, 26, 29, 30, 22, 21, 18, 17, 9, 10, 13, 14, 6, 5, 2, 1)


def kernel(x, Win0, Wout0, Win1, Wout1, Win2, Wout2):
    def body(x_ref, win0, wout0, win1, wout1, win2, wout2, out_ref,
             cxa, csa, cxb, csb, oxa, osa, oxb, osb,
             rxa, rsa, rxb, rsb, sxa, ssa, sxb, ssb,
             kxa, ksa, kxb, ksb):
        me = lax.axis_index("i")

        def cyc_at(idx):
            r = jnp.int32(CYC[N - 1])
            for j in range(N - 1):
                r = jnp.where(idx == j, jnp.int32(CYC[j]), r)
            return r

        q = jnp.int32(0)
        for j in range(N):
            q = jnp.where(me == CYC[j], jnp.int32(j), q)
        left = cyc_at(lax.rem(q + N - 1, N))
        right = cyc_at(lax.rem(q + 1, N))

        barrier = pltpu.get_barrier_semaphore()
        for nbr in (left, right):
            pl.semaphore_signal(barrier, inc=1, device_id=(nbr,),
                                device_id_type=MESH)
        pl.semaphore_wait(barrier, 2)

        def mlp(xc, win, wout):
            h = jnp.maximum(
                jax.lax.dot(xc, win[...],
                            preferred_element_type=jnp.float32),
                0.0)
            return jax.lax.dot(h, wout[...],
                               preferred_element_type=jnp.float32)

        def rc(src, dst, ssem, rsem, dev):
            return pltpu.make_async_remote_copy(
                src_ref=src, dst_ref=dst, send_sem=ssem, recv_sem=rsem,
                device_id=(dev,), device_id_type=MESH)

        def wait_recv(buf, slot, rsem):
            rc(buf.at[slot], buf.at[slot], sxa.at[0], rsem.at[slot],
               left).wait_recv()

        def wait_send(ssem):
            rc(oxa, oxa, ssem.at[0], rxa.at[0], right).wait_send()

        def signal(sem, dev):
            pl.semaphore_signal(sem, inc=1, device_id=(dev,),
                                device_id_type=MESH)

        weights = ((win0, wout0), (win1, wout1), (win2, wout2))

        for l in range(3):
            win, wout = weights[l]
            g0 = l * NHOP
            if l == 0:
                xav = x_ref[0:HB, :]
                xbv = x_ref[HB:BS, :]
            else:
                xav = csa[(g0 - 1) % K]
                xbv = csb[(g0 - 1) % K]
            p0 = mlp(jnp.concatenate([xav, xbv], axis=0), win, wout)
            if l > 0:
                pl.semaphore_wait(kxa, 1)
                pl.semaphore_wait(kxb, 1)
                pl.semaphore_wait(ksa, 1)
                pl.semaphore_wait(ksb, 1)
            oxa[...] = xav
            oxb[...] = xbv
            osa[...] = p0[0:HB, :]
            osb[...] = p0[HB:BS, :]
            sl0 = g0 % K
            rc(oxa, cxa.at[sl0], sxa.at[0], rxa.at[sl0], right).start()
            rc(oxb, cxb.at[sl0], sxb.at[0], rxb.at[sl0], left).start()
            rc(osa, csa.at[sl0], ssa.at[0], rsa.at[sl0], right).start()
            rc(osb, csb.at[sl0], ssb.at[0], rsb.at[sl0], left).start()

            def hop(s, _, win=win, wout=wout, l=l):
                g = l * NHOP + s
                slot = lax.rem(g, K)
                nslot = lax.rem(g + 1, K)

                wait_recv(cxa, slot, rxa)
                wait_recv(cxb, slot, rxb)
                wait_send(sxa)
                wait_send(sxb)

                def sig_x():
                    signal(kxa, left)
                    signal(kxb, right)
                if l == 0:
                    pl.when(s >= 1)(sig_x)
                elif l == 2:
                    pl.when(s <= 89 - 2 * NHOP)(sig_x)
                else:
                    sig_x()

                def fwd_x():
                    def go():
                        rc(cxa.at[slot], cxa.at[nslot], sxa.at[0],
                           rxa.at[nslot], right).start()
                        rc(cxb.at[slot], cxb.at[nslot], sxb.at[0],
                           rxb.at[nslot], left).start()
                    def kw():
                        pl.semaphore_wait(kxa, 1)
                        pl.semaphore_wait(kxb, 1)
                    if l == 0:
                        pl.when(s >= 3)(kw)
                    else:
                        kw()
                    go()
                pl.when(s < NHOP - 1)(fwd_x)

                p = mlp(jnp.concatenate([cxa[slot], cxb[slot]], axis=0),
                        win, wout)

                wait_recv(csa, slot, rsa)
                wait_recv(csb, slot, rsb)
                csa[slot] = csa[slot] + p[0:HB, :]
                csb[slot] = csb[slot] + p[HB:BS, :]

                wait_send(ssa)
                wait_send(ssb)

                def sig_s():
                    signal(ksa, left)
                    signal(ksb, right)
                if l == 0:
                    pl.when(s >= 1)(sig_s)
                else:
                    sig_s()

                def fwd_s():
                    def kw():
                        pl.semaphore_wait(ksa, 1)
                        pl.semaphore_wait(ksb, 1)
                    if l == 0:
                        pl.when(s >= 3)(kw)
                    else:
                        kw()
                    rc(csa.at[slot], csa.at[nslot], ssa.at[0],
                       rsa.at[nslot], right).start()
                    rc(csb.at[slot], csb.at[nslot], ssb.at[0],
                       rsb.at[nslot], left).start()
                pl.when(s < NHOP - 1)(fwd_s)
                return 0

            lax.fori_loop(0, NHOP, hop, 0)

        cma = cyc_at(lax.rem(q + 3, N))
        cmb = cyc_at(lax.rem(q - 3 + N, N))
        out_ref[pl.ds(cma * BS, HB), :] = csa[0]
        out_ref[pl.ds(cmb * BS + HB, HB), :] = csb[0]

        pl.semaphore_wait(ksa, 1)
        pl.semaphore_wait(ksb, 1)
        osa[...] = csa[0]
        osb[...] = csb[0]
        sl93 = 93 % K
        rc(osa, csa.at[sl93], ssa.at[0], rsa.at[sl93], right).start()
        rc(osb, csb.at[sl93], ssb.at[0], rsb.at[sl93], left).start()

        def ag(s, _):
            m = 93 + s
            slot = lax.rem(m, K)
            nslot = lax.rem(m + 1, K)
            wait_recv(csa, slot, rsa)
            wait_recv(csb, slot, rsb)
            wait_send(ssa)
            wait_send(ssb)
            def sig_s():
                signal(ksa, left)
                signal(ksb, right)
            pl.when(s <= 27)(sig_s)

            def fwd():
                pl.semaphore_wait(ksa, 1)
                pl.semaphore_wait(ksb, 1)
                rc(csa.at[slot], csa.at[nslot], ssa.at[0],
                   rsa.at[nslot], right).start()
                rc(csb.at[slot], csb.at[nslot], ssb.at[0],
                   rsb.at[nslot], left).start()
            pl.when(s < NHOP - 1)(fwd)

            ca = cyc_at(lax.rem(q + 2 - s + 2 * N, N))
            cb = cyc_at(lax.rem(q + s - 2 + 2 * N, N))
            out_ref[pl.ds(ca * BS, HB), :] = csa[slot]
            out_ref[pl.ds(cb * BS + HB, HB), :] = csb[slot]
            return 0

        lax.fori_loop(0, NHOP, ag, 0)

    return pl.pallas_call(
        body,
        out_shape=jax.ShapeDtypeStruct((N * BS, D), jnp.float32),
        in_specs=[pl.BlockSpec(memory_space=pltpu.VMEM)] * 7,
        out_specs=pl.BlockSpec(memory_space=pltpu.VMEM),
        scratch_shapes=[
            pltpu.VMEM((K, HB, D), jnp.float32),
            pltpu.VMEM((K, HB, D), jnp.float32),
            pltpu.VMEM((K, HB, D), jnp.float32),
            pltpu.VMEM((K, HB, D), jnp.float32),
            pltpu.VMEM((HB, D), jnp.float32),
            pltpu.VMEM((HB, D), jnp.float32),
            pltpu.VMEM((HB, D), jnp.float32),
            pltpu.VMEM((HB, D), jnp.float32),
            pltpu.SemaphoreType.DMA((K,)),
            pltpu.SemaphoreType.DMA((K,)),
            pltpu.SemaphoreType.DMA((K,)),
            pltpu.SemaphoreType.DMA((K,)),
            pltpu.SemaphoreType.DMA((1,)),
            pltpu.SemaphoreType.DMA((1,)),
            pltpu.SemaphoreType.DMA((1,)),
            pltpu.SemaphoreType.DMA((1,)),
            pltpu.SemaphoreType.REGULAR,
            pltpu.SemaphoreType.REGULAR,
            pltpu.SemaphoreType.REGULAR,
            pltpu.SemaphoreType.REGULAR,
        ],
        compiler_params=pltpu.CompilerParams(collective_id=0),
    )(x, Win0, Wout0, Win1, Wout1, Win2, Wout2)


# device time: 589163 ns/iter; 2.9542x vs baseline; 1.1715x over previous
import jax
import jax.numpy as jnp
from jax import lax
from jax.experimental import pallas as pl
from jax.experimental.pallas import tpu as pltpu

N = 32
BS = 512
HB = 256
D = 256
NHOP = N - 1
K = 4

MESH = pl.DeviceIdType.MESH

CYC = (0, 3, 4, 7, 15, 12, 11, 8, 16, 19, 20, 23, 31, 28, 27, 24,
       25, 26, 29, 30, 22, 21, 18, 17, 9, 10, 13, 14, 6, 5, 2, 1)


def kernel(x, Win0, Wout0, Win1, Wout1, Win2, Wout2):
    def body(x_ref, win0, wout0, win1, wout1, win2, wout2, out_ref,
             cxa, csa, cxb, csb, oxa, osa, oxb, osb,
             rxa, rsa, rxb, rsb, sxa, ssa, sxb, ssb,
             kxa, ksa, kxb, ksb):
        me = lax.axis_index("i")

        def cyc_at(idx):
            r = jnp.int32(CYC[N - 1])
            for j in range(N - 1):
                r = jnp.where(idx == j, jnp.int32(CYC[j]), r)
            return r

        q = jnp.int32(0)
        for j in range(N):
            q = jnp.where(me == CYC[j], jnp.int32(j), q)
        left = cyc_at(lax.rem(q + N - 1, N))
        right = cyc_at(lax.rem(q + 1, N))

        barrier = pltpu.get_barrier_semaphore()
        for nbr in (left, right):
            pl.semaphore_signal(barrier, inc=1, device_id=(nbr,),
                                device_id_type=MESH)
        pl.semaphore_wait(barrier, 2)

        def mlp(xc, win, wout):
            h = jnp.maximum(
                jax.lax.dot(xc.astype(jnp.float32), win[...],
                            preferred_element_type=jnp.float32),
                0.0)
            return jax.lax.dot(h, wout[...],
                               preferred_element_type=jnp.float32)

        def rc(src, dst, ssem, rsem, dev):
            return pltpu.make_async_remote_copy(
                src_ref=src, dst_ref=dst, send_sem=ssem, recv_sem=rsem,
                device_id=(dev,), device_id_type=MESH)

        def wait_recv(buf, slot, rsem):
            rc(buf.at[slot], buf.at[slot], sxa.at[0], rsem.at[slot],
               left).wait_recv()

        def wait_send(ssem, proto):
            rc(proto, proto, ssem.at[0], rxa.at[0], right).wait_send()

        def signal(sem, dev):
            pl.semaphore_signal(sem, inc=1, device_id=(dev,),
                                device_id_type=MESH)

        weights = ((win0, wout0), (win1, wout1), (win2, wout2))

        for l in range(3):
            win, wout = weights[l]
            g0 = l * NHOP
            if l == 0:
                xav = x_ref[0:HB, :]
                xbv = x_ref[HB:BS, :]
            else:
                xav = csa[(g0 - 1) % K]
                xbv = csb[(g0 - 1) % K]
            p0 = mlp(jnp.concatenate([xav, xbv], axis=0), win, wout)
            if l > 0:
                pl.semaphore_wait(kxa, 1)
                pl.semaphore_wait(kxb, 1)
                pl.semaphore_wait(ksa, 1)
                pl.semaphore_wait(ksb, 1)
            oxa[...] = xav.astype(jnp.bfloat16)
            oxb[...] = xbv.astype(jnp.bfloat16)
            osa[...] = p0[0:HB, :]
            osb[...] = p0[HB:BS, :]
            sl0 = g0 % K
            rc(oxa, cxa.at[sl0], sxa.at[0], rxa.at[sl0], right).start()
            rc(oxb, cxb.at[sl0], sxb.at[0], rxb.at[sl0], left).start()
            rc(osa, csa.at[sl0], ssa.at[0], rsa.at[sl0], right).start()
            rc(osb, csb.at[sl0], ssb.at[0], rsb.at[sl0], left).start()

            def hop(s, _, win=win, wout=wout, l=l):
                g = l * NHOP + s
                slot = lax.rem(g, K)
                nslot = lax.rem(g + 1, K)

                wait_recv(cxa, slot, rxa)
                wait_recv(cxb, slot, rxb)
                wait_send(sxa, oxa)
                wait_send(sxb, oxb)

                def sig_x():
                    signal(kxa, left)
                    signal(kxb, right)
                if l == 0:
                    pl.when(s >= 1)(sig_x)
                elif l == 2:
                    pl.when(s <= 89 - 2 * NHOP)(sig_x)
                else:
                    sig_x()

                def fwd_x():
                    def go():
                        rc(cxa.at[slot], cxa.at[nslot], sxa.at[0],
                           rxa.at[nslot], right).start()
                        rc(cxb.at[slot], cxb.at[nslot], sxb.at[0],
                           rxb.at[nslot], left).start()
                    def kw():
                        pl.semaphore_wait(kxa, 1)
                        pl.semaphore_wait(kxb, 1)
                    if l == 0:
                        pl.when(s >= 3)(kw)
                    else:
                        kw()
                    go()
                pl.when(s < NHOP - 1)(fwd_x)

                p = mlp(jnp.concatenate([cxa[slot], cxb[slot]], axis=0),
                        win, wout)

                wait_recv(csa, slot, rsa)
                wait_recv(csb, slot, rsb)
                csa[slot] = csa[slot] + p[0:HB, :]
                csb[slot] = csb[slot] + p[HB:BS, :]

                wait_send(ssa, osa)
                wait_send(ssb, osb)

                def sig_s():
                    signal(ksa, left)
                    signal(ksb, right)
                if l == 0:
                    pl.when(s >= 1)(sig_s)
                else:
                    sig_s()

                def fwd_s():
                    def kw():
                        pl.semaphore_wait(ksa, 1)
                        pl.semaphore_wait(ksb, 1)
                    if l == 0:
                        pl.when(s >= 3)(kw)
                    else:
                        kw()
                    rc(csa.at[slot], csa.at[nslot], ssa.at[0],
                       rsa.at[nslot], right).start()
                    rc(csb.at[slot], csb.at[nslot], ssb.at[0],
                       rsb.at[nslot], left).start()
                pl.when(s < NHOP - 1)(fwd_s)
                return 0

            lax.fori_loop(0, NHOP, hop, 0)

        cma = cyc_at(lax.rem(q + 3, N))
        cmb = cyc_at(lax.rem(q - 3 + N, N))
        out_ref[pl.ds(cma * BS, HB), :] = csa[0]
        out_ref[pl.ds(cmb * BS + HB, HB), :] = csb[0]

        pl.semaphore_wait(ksa, 1)
        pl.semaphore_wait(ksb, 1)
        osa[...] = csa[0]
        osb[...] = csb[0]
        sl93 = 93 % K
        rc(osa, csa.at[sl93], ssa.at[0], rsa.at[sl93], right).start()
        rc(osb, csb.at[sl93], ssb.at[0], rsb.at[sl93], left).start()

        def ag(s, _):
            m = 93 + s
            slot = lax.rem(m, K)
            nslot = lax.rem(m + 1, K)
            wait_recv(csa, slot, rsa)
            wait_recv(csb, slot, rsb)
            wait_send(ssa, osa)
            wait_send(ssb, osb)
            def sig_s():
                signal(ksa, left)
                signal(ksb, right)
            pl.when(s <= 27)(sig_s)

            def fwd():
                pl.semaphore_wait(ksa, 1)
                pl.semaphore_wait(ksb, 1)
                rc(csa.at[slot], csa.at[nslot], ssa.at[0],
                   rsa.at[nslot], right).start()
                rc(csb.at[slot], csb.at[nslot], ssb.at[0],
                   rsb.at[nslot], left).start()
            pl.when(s < NHOP - 1)(fwd)

            ca = cyc_at(lax.rem(q + 2 - s + 2 * N, N))
            cb = cyc_at(lax.rem(q + s - 2 + 2 * N, N))
            out_ref[pl.ds(ca * BS, HB), :] = csa[slot]
            out_ref[pl.ds(cb * BS + HB, HB), :] = csb[slot]
            return 0

        lax.fori_loop(0, NHOP, ag, 0)

    return pl.pallas_call(
        body,
        out_shape=jax.ShapeDtypeStruct((N * BS, D), jnp.float32),
        in_specs=[pl.BlockSpec(memory_space=pltpu.VMEM)] * 7,
        out_specs=pl.BlockSpec(memory_space=pltpu.VMEM),
        scratch_shapes=[
            pltpu.VMEM((K, HB, D), jnp.bfloat16),
            pltpu.VMEM((K, HB, D), jnp.float32),
            pltpu.VMEM((K, HB, D), jnp.bfloat16),
            pltpu.VMEM((K, HB, D), jnp.float32),
            pltpu.VMEM((HB, D), jnp.bfloat16),
            pltpu.VMEM((HB, D), jnp.float32),
            pltpu.VMEM((HB, D), jnp.bfloat16),
            pltpu.VMEM((HB, D), jnp.float32),
            pltpu.SemaphoreType.DMA((K,)),
            pltpu.SemaphoreType.DMA((K,)),
            pltpu.SemaphoreType.DMA((K,)),
            pltpu.SemaphoreType.DMA((K,)),
            pltpu.SemaphoreType.DMA((1,)),
            pltpu.SemaphoreType.DMA((1,)),
            pltpu.SemaphoreType.DMA((1,)),
            pltpu.SemaphoreType.DMA((1,)),
            pltpu.SemaphoreType.REGULAR,
            pltpu.SemaphoreType.REGULAR,
            pltpu.SemaphoreType.REGULAR,
            pltpu.SemaphoreType.REGULAR,
        ],
        compiler_params=pltpu.CompilerParams(collective_id=0),
    )(x, Win0, Wout0, Win1, Wout1, Win2, Wout2)
